# baseline (speedup 1.0000x reference)
import jax
import jax.numpy as jnp
from jax.experimental import pallas as pl
from jax.experimental.pallas import tpu as pltpu

_BN = 256


def _assign_body(x_ref, c_ref, prev_ref, idx_ref, eq_ref):
    i = pl.program_id(0)
    x = x_ref[...]
    c = c_ref[...]
    s = jax.lax.dot_general(
        x, c, (((1,), (1,)), ((), ())), preferred_element_type=jnp.float32
    )
    a2 = jnp.sum(x * x, axis=1, keepdims=True)
    b2 = jnp.sum(c * c, axis=1)
    d2 = jnp.maximum(a2 + b2[None, :] - 2.0 * s, 0.0)
    idx = jnp.argmin(d2, axis=1).astype(jnp.int32)
    idx_ref[0, 0, :] = idx
    eq = jnp.all(idx == prev_ref[0, 0, :])

    @pl.when(i == 0)
    def _():
        eq_ref[0, 0] = jnp.int32(1)

    eq_ref[0, 0] = eq_ref[0, 0] * eq.astype(jnp.int32)


def _assign(x, centroids, prev3):
    n, f = x.shape
    c, _ = centroids.shape
    nb = n // _BN
    return pl.pallas_call(
        _assign_body,
        grid=(nb,),
        in_specs=[
            pl.BlockSpec((_BN, f), lambda i: (i, 0)),
            pl.BlockSpec((c, f), lambda i: (0, 0)),
            pl.BlockSpec((1, 1, _BN), lambda i: (i, 0, 0)),
        ],
        out_specs=[
            pl.BlockSpec((1, 1, _BN), lambda i: (i, 0, 0)),
            pl.BlockSpec(memory_space=pltpu.SMEM),
        ],
        out_shape=[
            jax.ShapeDtypeStruct((nb, 1, _BN), jnp.int32),
            jax.ShapeDtypeStruct((1, 1), jnp.int32),
        ],
        compiler_params=pltpu.CompilerParams(
            dimension_semantics=("arbitrary",),
        ),
    )(x, centroids, prev3)


def _segsum_body(x_ref, a_ref, sums_ref, counts_ref):
    i = pl.program_id(0)

    @pl.when(i == 0)
    def _():
        sums_ref[...] = jnp.zeros_like(sums_ref)
        counts_ref[...] = jnp.zeros_like(counts_ref)

    a = a_ref[0, 0, :]
    nc = sums_ref.shape[0]
    one_hot = (
        a[:, None] == jax.lax.broadcasted_iota(jnp.int32, (a.shape[0], nc), 1)
    ).astype(jnp.float32)
    sums_ref[...] += jax.lax.dot_general(
        one_hot, x_ref[...], (((0,), (0,)), ((), ())),
        preferred_element_type=jnp.float32,
    )
    counts_ref[...] += jnp.sum(one_hot, axis=0, keepdims=True)


def _segsum(x, assign3, num_clusters):
    n, f = x.shape
    nb = n // _BN
    return pl.pallas_call(
        _segsum_body,
        grid=(nb,),
        in_specs=[
            pl.BlockSpec((_BN, f), lambda i: (i, 0)),
            pl.BlockSpec((1, 1, _BN), lambda i: (i, 0, 0)),
        ],
        out_specs=[
            pl.BlockSpec((num_clusters, f), lambda i: (0, 0)),
            pl.BlockSpec((1, num_clusters), lambda i: (0, 0)),
        ],
        out_shape=[
            jax.ShapeDtypeStruct((num_clusters, f), jnp.float32),
            jax.ShapeDtypeStruct((1, num_clusters), jnp.float32),
        ],
        compiler_params=pltpu.CompilerParams(
            dimension_semantics=("arbitrary",),
        ),
    )(x, assign3)


def kernel(x, prev_assignments, centroids):
    n, f = x.shape
    num_clusters = centroids.shape[0]
    nb = n // _BN
    prev3 = prev_assignments.reshape(nb, 1, _BN)

    idx3, eqf = _assign(x, centroids, prev3)
    new_assignments = idx3.reshape(n)
    pred = eqf[0, 0] == 1

    decay = jnp.float32(0.99)

    def _upd(ops):
        x_, c_, assign_ = ops
        sums, counts = _segsum(x_, assign_.reshape(nb, 1, _BN), num_clusters)
        new_centroids = sums / counts[0][:, None]
        updated = (1.0 - decay) * c_ + decay * new_centroids
        idx2, _ = _assign(x_, updated, prev3)
        return idx2.reshape(n), jnp.array(True)

    def _keep(ops):
        return ops[2], jnp.array(False)

    return jax.lax.cond(pred, _upd, _keep, (x, centroids, new_assignments))


# speedup vs baseline: 1.2221x; 1.2221x over previous
import jax
import jax.numpy as jnp
from jax.experimental import pallas as pl
from jax.experimental.pallas import tpu as pltpu

_BN = 512
_H = 256


def _assign_body(x_ref, c_ref, cts_ref, prev_ref, idx_ref, eq_ref, b2_ref):
    i = pl.program_id(0)

    @pl.when(i == 0)
    def _():
        c = c_ref[...]
        b2_ref[...] = jnp.sum(c * c, axis=1)[None, :]
        eq_ref[0, 0] = jnp.int32(1)

    b2 = b2_ref[...]
    eq_all = None
    for h in range(2):
        x = x_ref[h * _H:(h + 1) * _H, :]
        s = jnp.dot(x, cts_ref[...], preferred_element_type=jnp.float32)
        a2 = jnp.sum(x * x, axis=1, keepdims=True)
        d2 = jnp.maximum(a2 + b2 + s, 0.0)
        idx = jnp.argmin(d2, axis=1).astype(jnp.int32)
        idx_ref[0, h, :] = idx
        eq = jnp.all(idx == prev_ref[0, h, :])
        eq_all = eq if eq_all is None else (eq_all & eq)

    eq_ref[0, 0] = eq_ref[0, 0] * eq_all.astype(jnp.int32)


def _assign(x, centroids, cts, prev3):
    n, f = x.shape
    c, _ = centroids.shape
    nb = n // _BN
    return pl.pallas_call(
        _assign_body,
        grid=(nb,),
        in_specs=[
            pl.BlockSpec((_BN, f), lambda i: (i, 0)),
            pl.BlockSpec((c, f), lambda i: (0, 0)),
            pl.BlockSpec((f, c), lambda i: (0, 0)),
            pl.BlockSpec((1, 2, _H), lambda i: (i, 0, 0)),
        ],
        out_specs=[
            pl.BlockSpec((1, 2, _H), lambda i: (i, 0, 0)),
            pl.BlockSpec(memory_space=pltpu.SMEM),
        ],
        out_shape=[
            jax.ShapeDtypeStruct((nb, 2, _H), jnp.int32),
            jax.ShapeDtypeStruct((1, 1), jnp.int32),
        ],
        scratch_shapes=[pltpu.VMEM((1, c), jnp.float32)],
        compiler_params=pltpu.CompilerParams(
            dimension_semantics=("arbitrary",),
        ),
    )(x, centroids, cts, prev3)


def _segsum_body(x_ref, a_ref, sums_ref, counts_ref):
    i = pl.program_id(0)

    @pl.when(i == 0)
    def _():
        sums_ref[...] = jnp.zeros_like(sums_ref)
        counts_ref[...] = jnp.zeros_like(counts_ref)

    a = a_ref[0, 0, :]
    nc = sums_ref.shape[0]
    one_hot = (
        a[:, None] == jax.lax.broadcasted_iota(jnp.int32, (a.shape[0], nc), 1)
    ).astype(jnp.float32)
    sums_ref[...] += jax.lax.dot_general(
        one_hot, x_ref[...], (((0,), (0,)), ((), ())),
        preferred_element_type=jnp.float32,
    )
    counts_ref[...] += jnp.sum(one_hot, axis=0, keepdims=True)


def _segsum(x, assign3, num_clusters):
    n, f = x.shape
    nb = n // _BN
    return pl.pallas_call(
        _segsum_body,
        grid=(nb,),
        in_specs=[
            pl.BlockSpec((_BN, f), lambda i: (i, 0)),
            pl.BlockSpec((1, 1, _BN), lambda i: (i, 0, 0)),
        ],
        out_specs=[
            pl.BlockSpec((num_clusters, f), lambda i: (0, 0)),
            pl.BlockSpec((1, num_clusters), lambda i: (0, 0)),
        ],
        out_shape=[
            jax.ShapeDtypeStruct((num_clusters, f), jnp.float32),
            jax.ShapeDtypeStruct((1, num_clusters), jnp.float32),
        ],
        compiler_params=pltpu.CompilerParams(
            dimension_semantics=("arbitrary",),
        ),
    )(x, assign3)


def kernel(x, prev_assignments, centroids):
    n, f = x.shape
    num_clusters = centroids.shape[0]
    nb = n // _BN
    prev3 = prev_assignments.reshape(nb, 2, _H)
    cts = (-2.0 * centroids).T

    idx3, eqf = _assign(x, centroids, cts, prev3)
    new_assignments = idx3.reshape(n)
    pred = eqf[0, 0] == 1

    decay = jnp.float32(0.99)

    def _upd(ops):
        x_, c_, assign_ = ops
        sums, counts = _segsum(x_, assign_.reshape(nb, 1, _BN), num_clusters)
        new_centroids = sums / counts[0][:, None]
        updated = (1.0 - decay) * c_ + decay * new_centroids
        cts2 = (-2.0 * updated).T
        idx2, _ = _assign(x_, updated, cts2, prev3)
        return idx2.reshape(n), jnp.array(True)

    def _keep(ops):
        return ops[2], jnp.array(False)

    return jax.lax.cond(pred, _upd, _keep, (x, centroids, new_assignments))
